# Initial kernel scaffold; baseline (speedup 1.0000x reference)
import jax
import jax.numpy as jnp
from jax.experimental import pallas as pl


def kernel(coords, features, coords_list_0, in_features_0, coords_list_1, in_features_1, coords_list_2, in_features_2, coords_list_3, in_features_3, coords_list_4, in_features_4, s0_fp0_w, s0_fp0_b, s0_blk0_pw, s0_blk0_pb, s1_fp0_w, s1_fp0_b, s1_blk0_pw, s1_blk0_pb, s2_fp0_w, s2_fp0_b, s2_fp1_w, s2_fp1_b, s2_blk0_pw, s2_blk0_pb, s3_fp0_w, s3_fp0_b, s3_fp1_w, s3_fp1_b, s3_blk0_pw, s3_blk0_pb, s3_blk0_vw, s3_blk0_vb, s3_blk1_pw, s3_blk1_pb, s3_blk1_vw, s3_blk1_vb, s4_fp0_w, s4_fp0_b, s4_fp1_w, s4_fp1_b, s4_fp2_w, s4_fp2_b, s4_blk0_pw, s4_blk0_pb, s4_blk0_vw, s4_blk0_vb, cls0_w, cls0_b, cls1_w, cls1_b, rec0_w, rec0_b, rec1_w, rec1_b, rec2_w, rec2_b):
    raise NotImplementedError("write your pallas kernel here")


# speedup vs baseline: 11.3451x; 11.3451x over previous
import functools

import jax
import jax.numpy as jnp
from jax import lax
from jax.experimental import pallas as pl
from jax.experimental.pallas import tpu as pltpu, tpu_sc as plsc

_F32 = jnp.float32
_B = 4


def _mm(a, b):
    return jnp.dot(a, b, preferred_element_type=_F32)


def _interp_matrix(d2, nc):
    lane = lax.broadcasted_iota(jnp.int32, d2.shape, 1)
    cur = d2
    idxs, dists = [], []
    for k in range(3):
        mval = jnp.min(cur, axis=1, keepdims=True)
        midx = jnp.min(jnp.where(cur == mval, lane, nc), axis=1, keepdims=True)
        idxs.append(midx)
        dists.append(jnp.maximum(mval, 0.0))
        if k < 2:
            cur = jnp.where(lane == midx, _F32(jnp.inf), cur)
    ws = [1.0 / (d + 1e-8) for d in dists]
    tot = ws[0] + ws[1] + ws[2]
    s = jnp.zeros_like(d2)
    for k in range(3):
        s = s + jnp.where(lane == idxs[k], ws[k] / tot, 0.0)
    return s


def _head_body(nc, nblk, nlayers, wpad, *refs):
    dense = refs[0][0]
    coarse3 = refs[1][0]
    featsT = refs[2][0]
    skipT = refs[3][0]
    w0a, w0b, b0 = refs[4], refs[5], refs[6]
    dn = jnp.sum(dense * dense, axis=1, keepdims=True)
    cn = jnp.sum(coarse3 * coarse3, axis=0, keepdims=True)
    d2 = dn + cn - 2.0 * _mm(dense, coarse3)
    s = _interp_matrix(d2, nc)
    interp = _mm(s, featsT)
    x = jnp.maximum(_mm(interp, w0a[...]) + _mm(skipT, w0b[...]) + b0[...], 0.0)
    for i in range(nlayers):
        w, b = refs[7 + 2 * i], refs[8 + 2 * i]
        x = jnp.maximum(_mm(x, w[...]) + b[...], 0.0)
    out_x = refs[7 + 2 * nlayers]
    out_x[0] = x
    if wpad:
        n, co = x.shape
        ones = jnp.ones((n, 1), _F32)
        zeros = jnp.zeros((n, wpad - co - 1), _F32)
        refs[8 + 2 * nlayers][0] = jnp.concatenate([x, ones, zeros], axis=1)


def _run_head(denseT, coarse3, featsT, skipT, w0, b0, layers, nblk, wpad):
    b, nd, _ = denseT.shape
    nc = coarse3.shape[2]
    cfeat = featsT.shape[2]
    blk = nd // nblk
    g = b * nblk
    denseT = denseT.reshape(g, blk, 3)
    skipT = skipT.reshape(g, blk, skipT.shape[2])
    w0a = jnp.transpose(w0[:, :cfeat])
    w0b = jnp.transpose(w0[:, cfeat:])
    co_last = (layers[-1][0] if layers else w0).shape[0]

    def bmap(_g):
        return (_g // nblk, 0, 0)

    in_specs = [
        pl.BlockSpec((1, blk, 3), lambda i: (i, 0, 0)),
        pl.BlockSpec((1, 3, nc), bmap),
        pl.BlockSpec((1, nc, cfeat), bmap),
        pl.BlockSpec((1, blk, skipT.shape[2]), lambda i: (i, 0, 0)),
        pl.BlockSpec(w0a.shape, lambda i: (0, 0)),
        pl.BlockSpec(w0b.shape, lambda i: (0, 0)),
        pl.BlockSpec(b0.shape, lambda i: (0,)),
    ]
    args = [denseT, coarse3, featsT, skipT, w0a, w0b, b0]
    for (w, bb) in layers:
        args += [jnp.transpose(w), bb]
        in_specs += [pl.BlockSpec((w.shape[1], w.shape[0]), lambda i: (0, 0)),
                     pl.BlockSpec(bb.shape, lambda i: (0,))]
    out_shape = [jax.ShapeDtypeStruct((g, blk, co_last), _F32)]
    out_specs = [pl.BlockSpec((1, blk, co_last), lambda i: (i, 0, 0))]
    if wpad:
        out_shape.append(jax.ShapeDtypeStruct((g, blk, wpad), _F32))
        out_specs.append(pl.BlockSpec((1, blk, wpad), lambda i: (i, 0, 0)))
    outs = pl.pallas_call(
        functools.partial(_head_body, nc, nblk, len(layers), wpad),
        grid=(g,),
        in_specs=in_specs,
        out_specs=out_specs,
        out_shape=out_shape,
    )(*args)
    x = outs[0].reshape(b, nd, co_last)
    if wpad:
        return x, outs[1].reshape(b * nd, wpad)
    return x


def _vid_body(r, v, nparts, dense3_ref, out_ref, loc_ref):
    d3 = dense3_ref[0]
    cmin = jnp.min(d3, axis=1, keepdims=True)
    cmax = jnp.max(d3, axis=1, keepdims=True)
    norm = (d3 - cmin) / (cmax - cmin + 1e-8) * (r - 1.0)
    vid = jnp.clip(jnp.round(norm), 0.0, r - 1.0).astype(jnp.int32)
    flat = vid[0:1, :] * (r * r) + vid[1:2, :] * r + vid[2:3, :]
    bi = pl.program_id(0)
    out_ref[0] = flat + bi * v
    vp = v // nparts
    for p in range(nparts):
        lid = flat - p * vp
        ok = (lid >= 0) & (lid < vp)
        loc_ref[p, 0] = jnp.where(ok, lid, vp)


def _run_vid(dense3, r, nparts):
    b, _, nd = dense3.shape
    v = r * r * r
    out, loc = pl.pallas_call(
        functools.partial(_vid_body, r, v, nparts),
        grid=(b,),
        in_specs=[pl.BlockSpec((1, 3, nd), lambda i: (i, 0, 0))],
        out_specs=[pl.BlockSpec((1, 1, nd), lambda i: (i, 0, 0)),
                   pl.BlockSpec((nparts, 1, 1, nd), lambda i: (0, i, 0, 0))],
        out_shape=[jax.ShapeDtypeStruct((b, 1, nd), jnp.int32),
                   jax.ShapeDtypeStruct((nparts, b, 1, nd), jnp.int32)],
    )(dense3)
    return (out.reshape(b * nd // 128, 128),
            loc.reshape(nparts * b * nd // 128, 128))


def _sc_voxelize(rows, loc2, nb, nd, v, nparts, with_counts):
    wp = 128
    vp = v // nparts
    hv = vp + 128
    zr = hv // 16
    npts = nd // 16
    nch = npts // 128
    idxrows = nd // 128
    wr = vp // 16
    nq = nparts // 2
    mesh = plsc.VectorSubcoreMesh(core_axis_name="c", subcore_axis_name="s")

    out_type = [jax.ShapeDtypeStruct((nb * v, wp), _F32)]
    scratch = [
        pltpu.VMEM((zr, wp), _F32),
        pltpu.VMEM((idxrows, 128), jnp.int32),
        pltpu.VMEM((128, wp), _F32),
        pltpu.VMEM_SHARED((hv, wp), _F32),
    ]
    if with_counts:
        out_type.append(jax.ShapeDtypeStruct((nb * v, wp), _F32))
        scratch += [pltpu.VMEM((128, wp), _F32),
                    pltpu.VMEM_SHARED((hv, wp), _F32)]

    @functools.partial(pl.kernel, out_type=tuple(out_type), mesh=mesh,
                       scratch_types=scratch)
    def k(rows_hbm, loc_hbm, *rest):
        if with_counts:
            out_hbm, cout_hbm, zbuf, idxv, rowsv, spmem, cbuf, cspmem = rest
        else:
            out_hbm, zbuf, idxv, rowsv, spmem = rest
        c = lax.axis_index("c")
        s = lax.axis_index("s")
        zvec = jnp.zeros((16,), _F32)

        def zero_body(i, _):
            for j in range(wp // 16):
                zbuf[i, pl.ds(j * 16, 16)] = zvec
            return _

        lax.fori_loop(0, zr, zero_body, 0)
        if with_counts:
            onevec = jnp.where(lax.iota(jnp.int32, 16) == 0, 1.0, 0.0)

            def cfill_body(i, _):
                cbuf[i, pl.ds(0, 16)] = onevec
                for j in range(1, wp // 16):
                    cbuf[i, pl.ds(j * 16, 16)] = zvec
                return _

            lax.fori_loop(0, 128, cfill_body, 0)
        for b in range(nb):
            for q in range(nq):
                part = q * 2 + c
                pltpu.sync_copy(zbuf, spmem.at[pl.ds(s * zr, zr)])
                if with_counts:
                    pltpu.sync_copy(zbuf, cspmem.at[pl.ds(s * zr, zr)])
                plsc.subcore_barrier()
                pltpu.sync_copy(
                    loc_hbm.at[pl.ds((part * nb + b) * idxrows, idxrows)],
                    idxv)
                base = b * nd + s * npts
                for j in range(nch):
                    pltpu.sync_copy(rows_hbm.at[pl.ds(base + j * 128, 128)],
                                    rowsv)
                    pltpu.sync_copy(rowsv, spmem.at[idxv.at[s * nch + j]],
                                    add=True)
                    if with_counts:
                        pltpu.sync_copy(cbuf,
                                        cspmem.at[idxv.at[s * nch + j]],
                                        add=True)
                plsc.subcore_barrier()
                pltpu.sync_copy(
                    spmem.at[pl.ds(s * wr, wr)],
                    out_hbm.at[pl.ds(b * v + part * vp + s * wr, wr)])
                if with_counts:
                    pltpu.sync_copy(
                        cspmem.at[pl.ds(s * wr, wr)],
                        cout_hbm.at[pl.ds(b * v + part * vp + s * wr, wr)])
                plsc.subcore_barrier()

    return k(rows, loc2)


def _sc_gather(table, flat2, m, cw):
    mpw = m // 32
    nch = mpw // 128
    idxrows = m // 128
    mesh = plsc.VectorSubcoreMesh(core_axis_name="c", subcore_axis_name="s")

    @functools.partial(
        pl.kernel,
        out_type=jax.ShapeDtypeStruct((m, cw), _F32),
        mesh=mesh,
        scratch_types=[
            pltpu.VMEM((idxrows, 128), jnp.int32),
            pltpu.VMEM((128, cw), _F32),
            pltpu.SemaphoreType.DMA,
        ],
    )
    def k(table_hbm, flat_hbm, out_hbm, idxv, rowsv, sem):
        c = lax.axis_index("c")
        s = lax.axis_index("s")
        w = s * 2 + c
        pltpu.sync_copy(flat_hbm, idxv)
        for j in range(nch):
            pltpu.async_copy(table_hbm.at[idxv.at[w * nch + j]], rowsv,
                             sem).wait()
            pltpu.sync_copy(rowsv, out_hbm.at[pl.ds(w * mpw + j * 128, 128)])

    return k(table, flat2)


def _roll_rows(a, sft):
    if sft == 0:
        return a
    n = a.shape[0]
    sft = sft % n
    return jnp.concatenate([a[sft:], a[:sft]], axis=0)


def _conv_body(r, ci, co, cnt_col, sep, *refs):
    r2 = r * r
    x = pl.program_id(1)
    nslab = 6 if sep else 3
    wt_ref, vb_ref, out_ref = refs[nslab], refs[nslab + 1], refs[nslab + 2]

    def mean_of(i):
        sl = refs[i][0, 0]
        cnt = refs[i + 3][0, 0] if sep else sl
        return sl[:, :ci] / jnp.maximum(cnt[:, cnt_col:cnt_col + 1], 1.0)

    fprev = mean_of(0) * jnp.where(x > 0, 1.0, 0.0)
    fcur = mean_of(1)
    fnxt = mean_of(2) * jnp.where(x < r - 1, 1.0, 0.0)
    row = lax.broadcasted_iota(jnp.int32, (r2, 1), 0)
    y = row // r
    z = row - y * r
    masks = {
        (-1, 0): (y > 0).astype(_F32), (1, 0): (y < r - 1).astype(_F32),
        (0, -1): (z > 0).astype(_F32), (0, 1): (z < r - 1).astype(_F32),
    }
    acc = jnp.zeros((r2, co), _F32)
    k = 0
    for src in (fprev, fcur, fnxt):
        for dy in (-1, 0, 1):
            for dz in (-1, 0, 1):
                sh = _roll_rows(src, dy * r + dz)
                if dy != 0:
                    sh = sh * masks[(dy, 0)]
                if dz != 0:
                    sh = sh * masks[(0, dz)]
                acc = acc + _mm(sh, wt_ref[k])
                k += 1
    out_ref[0, 0] = jnp.maximum(acc + vb_ref[...], 0.0)


def _run_conv(vox, cnt, cnt_col, vw, vb, r, nb, co_pad):
    co, ci = vw.shape[0], vw.shape[1]
    wp = vox.shape[1]
    r2 = r * r
    vox4 = vox.reshape(nb, r, r2, wp)
    sep = cnt is not None
    wt = jnp.transpose(vw, (2, 3, 4, 1, 0)).reshape(27, ci, co)
    if co_pad > co:
        wt = jnp.concatenate(
            [wt, jnp.zeros((27, ci, co_pad - co), _F32)], axis=2)
        vb = jnp.concatenate([vb, jnp.zeros((co_pad - co,), _F32)])

    def clampm(d):
        def im(b, x):
            return (b, jnp.clip(x + d, 0, r - 1), 0, 0)
        return im

    slab = lambda w: [pl.BlockSpec((1, 1, r2, w), clampm(-1)),
                      pl.BlockSpec((1, 1, r2, w), clampm(0)),
                      pl.BlockSpec((1, 1, r2, w), clampm(1))]
    in_specs = slab(wp)
    args = [vox4, vox4, vox4]
    if sep:
        cnt4 = cnt.reshape(nb, r, r2, cnt.shape[1])
        in_specs += slab(cnt.shape[1])
        args += [cnt4, cnt4, cnt4]
    in_specs += [pl.BlockSpec((27, ci, co_pad), lambda b, x: (0, 0, 0)),
                 pl.BlockSpec((co_pad,), lambda b, x: (0,))]
    args += [wt, vb]
    out = pl.pallas_call(
        functools.partial(_conv_body, r, ci, co_pad, cnt_col, sep),
        grid=(nb, r),
        in_specs=in_specs,
        out_specs=pl.BlockSpec((1, 1, r2, co_pad), lambda b, x: (b, x, 0, 0)),
        out_shape=jax.ShapeDtypeStruct((nb, r, r2, co_pad), _F32),
    )(*args)
    return out.reshape(nb * r * r2, co_pad)


def _combine_body(wpad, nhead, *refs):
    x = refs[0][0]
    g = refs[1][0]
    pwt, pb = refs[2], refs[3]
    y = g + jnp.maximum(_mm(x, pwt[...]) + pb[...], 0.0)
    out_x = refs[4 + 2 * nhead]
    out_x[0] = y
    if wpad:
        n, co = y.shape
        ones = jnp.ones((n, 1), _F32)
        zeros = jnp.zeros((n, wpad - co - 1), _F32)
        refs[5 + 2 * nhead][0] = jnp.concatenate([y, ones, zeros], axis=1)
    if nhead:
        c0w, c0b = refs[4], refs[5]
        c1w, c1b = refs[6], refs[7]
        r0w, r0b = refs[8], refs[9]
        r1w, r1b = refs[10], refs[11]
        r2w, r2b = refs[12], refs[13]
        h = jnp.maximum(_mm(y, c0w[...]) + c0b[...], 0.0)
        seg = _mm(h, c1w[...]) + c1b[...]
        gg = jnp.maximum(_mm(y, r0w[...]) + r0b[...], 0.0)
        gg = jnp.maximum(_mm(gg, r1w[...]) + r1b[...], 0.0)
        rec = _mm(gg, r2w[...]) + r2b[...]
        refs[5 + 2 * nhead][0] = seg
        refs[6 + 2 * nhead][0] = rec


def _run_combine(x, g, pw, pb, wpad=0, heads=None):
    b, nd, ci = x.shape
    co = pw.shape[0]
    g3 = g.reshape(b, nd, co)
    args = [x, g3, jnp.transpose(pw), pb]
    in_specs = [
        pl.BlockSpec((1, nd, ci), lambda i: (i, 0, 0)),
        pl.BlockSpec((1, nd, co), lambda i: (i, 0, 0)),
        pl.BlockSpec((ci, co), lambda i: (0, 0)),
        pl.BlockSpec((co,), lambda i: (0,)),
    ]
    nhead = 0
    if heads is not None:
        nhead = 5
        for (w, bb) in heads:
            args += [jnp.transpose(w), bb]
            in_specs += [pl.BlockSpec((w.shape[1], w.shape[0]), lambda i: (0, 0)),
                         pl.BlockSpec(bb.shape, lambda i: (0,))]
    out_shape = [jax.ShapeDtypeStruct((b, nd, co), _F32)]
    out_specs = [pl.BlockSpec((1, nd, co), lambda i: (i, 0, 0))]
    if wpad:
        out_shape.append(jax.ShapeDtypeStruct((b, nd, wpad), _F32))
        out_specs.append(pl.BlockSpec((1, nd, wpad), lambda i: (i, 0, 0)))
    if heads is not None:
        nseg = heads[1][0].shape[0]
        nrec = heads[4][0].shape[0]
        out_shape += [jax.ShapeDtypeStruct((b, nd, nseg), _F32),
                      jax.ShapeDtypeStruct((b, nd, nrec), _F32)]
        out_specs += [pl.BlockSpec((1, nd, nseg), lambda i: (i, 0, 0)),
                      pl.BlockSpec((1, nd, nrec), lambda i: (i, 0, 0))]
    outs = pl.pallas_call(
        functools.partial(_combine_body, wpad, nhead),
        grid=(b,),
        in_specs=in_specs,
        out_specs=out_specs,
        out_shape=out_shape,
    )(*args)
    outs = list(outs)
    if wpad:
        outs[1] = outs[1].reshape(b * nd, wpad)
    return outs


def kernel(coords, features, coords_list_0, in_features_0, coords_list_1,
           in_features_1, coords_list_2, in_features_2, coords_list_3,
           in_features_3, coords_list_4, in_features_4,
           s0_fp0_w, s0_fp0_b, s0_blk0_pw, s0_blk0_pb,
           s1_fp0_w, s1_fp0_b, s1_blk0_pw, s1_blk0_pb,
           s2_fp0_w, s2_fp0_b, s2_fp1_w, s2_fp1_b, s2_blk0_pw, s2_blk0_pb,
           s3_fp0_w, s3_fp0_b, s3_fp1_w, s3_fp1_b,
           s3_blk0_pw, s3_blk0_pb, s3_blk0_vw, s3_blk0_vb,
           s3_blk1_pw, s3_blk1_pb, s3_blk1_vw, s3_blk1_vb,
           s4_fp0_w, s4_fp0_b, s4_fp1_w, s4_fp1_b, s4_fp2_w, s4_fp2_b,
           s4_blk0_pw, s4_blk0_pb, s4_blk0_vw, s4_blk0_vb,
           cls0_w, cls0_b, cls1_w, cls1_b,
           rec0_w, rec0_b, rec1_w, rec1_b, rec2_w, rec2_b):
    tt = lambda a: jnp.transpose(a, (0, 2, 1))
    x0 = _run_head(tt(coords_list_4), coords, tt(features), tt(in_features_4),
                   s0_fp0_w, s0_fp0_b, [(s0_blk0_pw, s0_blk0_pb)], 1, 0)
    x1 = _run_head(tt(coords_list_3), coords_list_4, x0, tt(in_features_3),
                   s1_fp0_w, s1_fp0_b, [(s1_blk0_pw, s1_blk0_pb)], 1, 0)
    x2 = _run_head(tt(coords_list_2), coords_list_3, x1, tt(in_features_2),
                   s2_fp0_w, s2_fp0_b,
                   [(s2_fp1_w, s2_fp1_b), (s2_blk0_pw, s2_blk0_pb)], 1, 0)
    x3 = _run_head(tt(coords_list_1), coords_list_2, x2,
                   tt(in_features_1), s3_fp0_w, s3_fp0_b,
                   [(s3_fp1_w, s3_fp1_b)], 1, 0)
    flat3, loc3 = _run_vid(coords_list_1, 16, 4)
    vox, cnt3 = _sc_voxelize(x3.reshape(_B * 2048, 128), loc3,
                             _B, 2048, 4096, 4, True)
    conv = _run_conv(vox, cnt3, 0, s3_blk0_vw, s3_blk0_vb, 16, _B, 128)
    g = _sc_gather(conv, flat3, _B * 2048, 128)
    (x3,) = _run_combine(x3, g, s3_blk0_pw, s3_blk0_pb)
    vox, cnt3 = _sc_voxelize(x3.reshape(_B * 2048, 128), loc3,
                             _B, 2048, 4096, 4, True)
    conv = _run_conv(vox, cnt3, 0, s3_blk1_vw, s3_blk1_vb, 16, _B, 128)
    g = _sc_gather(conv, flat3, _B * 2048, 128)
    (x3,) = _run_combine(x3, g, s3_blk1_pw, s3_blk1_pb)
    x4, rows4 = _run_head(tt(coords_list_0), coords_list_1, x3,
                          tt(in_features_0), s4_fp0_w, s4_fp0_b,
                          [(s4_fp1_w, s4_fp1_b), (s4_fp2_w, s4_fp2_b)], 16, 128)
    flat4, loc4 = _run_vid(coords_list_0, 32, 8)
    (vox,) = _sc_voxelize(rows4, loc4, _B, 8192, 32768, 8, False)
    conv = _run_conv(vox, None, 64, s4_blk0_vw, s4_blk0_vb, 32, _B, 128)
    g = _sc_gather(conv, flat4, _B * 8192, 128)
    g = g[:, :64]
    _, seg, rec = _run_combine(
        x4, g, s4_blk0_pw, s4_blk0_pb, wpad=0,
        heads=[(cls0_w, cls0_b), (cls1_w, cls1_b), (rec0_w, rec0_b),
               (rec1_w, rec1_b), (rec2_w, rec2_b)])
    return tt(seg), tt(rec)


# speedup vs baseline: 13.6740x; 1.2053x over previous
import functools

import jax
import jax.numpy as jnp
from jax import lax
from jax.experimental import pallas as pl
from jax.experimental.pallas import tpu as pltpu, tpu_sc as plsc

_F32 = jnp.float32
_B = 4


def _mm(a, b):
    return jnp.dot(a, b, preferred_element_type=_F32)


def _mmb(a, b):
    return jnp.dot(a.astype(jnp.bfloat16), b.astype(jnp.bfloat16),
                   preferred_element_type=_F32)


def _interp_matrix(d2, nc):
    lane = lax.broadcasted_iota(jnp.int32, d2.shape, 1)
    cur = d2
    idxs, dists = [], []
    for k in range(3):
        mval = jnp.min(cur, axis=1, keepdims=True)
        midx = jnp.min(jnp.where(cur == mval, lane, nc), axis=1, keepdims=True)
        idxs.append(midx)
        dists.append(jnp.maximum(mval, 0.0))
        if k < 2:
            cur = jnp.where(lane == midx, _F32(jnp.inf), cur)
    ws = [1.0 / (d + 1e-8) for d in dists]
    tot = ws[0] + ws[1] + ws[2]
    s = jnp.zeros_like(d2)
    for k in range(3):
        s = s + jnp.where(lane == idxs[k], ws[k] / tot, 0.0)
    return s


def _nn_body(nc, nblk, dense_ref, coarse3_ref, idx_ref, d_ref):
    dense = dense_ref[0]
    coarse3 = coarse3_ref[0]
    dn = jnp.sum(dense * dense, axis=1, keepdims=True)
    cn = jnp.sum(coarse3 * coarse3, axis=0, keepdims=True)
    d2 = dn + cn - 2.0 * _mm(dense, coarse3)
    lane = lax.broadcasted_iota(jnp.int32, d2.shape, 1)
    cur = d2
    idxs, dists = [], []
    for k in range(3):
        mval = jnp.min(cur, axis=1, keepdims=True)
        midx = jnp.min(jnp.where(cur == mval, lane, nc), axis=1, keepdims=True)
        idxs.append(midx)
        dists.append(mval)
        if k < 2:
            cur = jnp.where(lane == midx, _F32(jnp.inf), cur)
    idx_ref[0] = jnp.concatenate(idxs, axis=1)
    d_ref[0] = jnp.concatenate(dists, axis=1)


def _run_nn(denseT, coarse3, nblk):
    b, nd, _ = denseT.shape
    nc = coarse3.shape[2]
    blk = nd // nblk
    g = b * nblk
    denseT = denseT.reshape(g, blk, 3)
    idx, d = pl.pallas_call(
        functools.partial(_nn_body, nc, nblk),
        grid=(g,),
        in_specs=[pl.BlockSpec((1, blk, 3), lambda i: (i, 0, 0)),
                  pl.BlockSpec((1, 3, nc), lambda i: (i // nblk, 0, 0))],
        out_specs=[pl.BlockSpec((1, blk, 3), lambda i: (i, 0, 0)),
                   pl.BlockSpec((1, blk, 3), lambda i: (i, 0, 0))],
        out_shape=[jax.ShapeDtypeStruct((g, blk, 3), jnp.int32),
                   jax.ShapeDtypeStruct((g, blk, 3), _F32)],
    )(denseT, coarse3)
    return idx, d


def _head_body(nc, nblk, nlayers, wpad, bf, nn, *refs):
    featsT = refs[2][0]
    skipT = refs[3][0]
    w0a, w0b, b0 = refs[4], refs[5], refs[6]
    if nn:
        idxr = refs[0][0]
        dr = refs[1][0]
        lane = lax.broadcasted_iota(jnp.int32, (idxr.shape[0], nc), 1)
        ws = [1.0 / (jnp.maximum(dr[:, k:k + 1], 0.0) + 1e-8)
              for k in range(3)]
        tot = ws[0] + ws[1] + ws[2]
        s = jnp.zeros((idxr.shape[0], nc), _F32)
        for k in range(3):
            s = s + jnp.where(lane == idxr[:, k:k + 1], ws[k] / tot, 0.0)
    else:
        dense = refs[0][0]
        coarse3 = refs[1][0]
        dn = jnp.sum(dense * dense, axis=1, keepdims=True)
        cn = jnp.sum(coarse3 * coarse3, axis=0, keepdims=True)
        d2 = dn + cn - 2.0 * _mm(dense, coarse3)
        s = _interp_matrix(d2, nc)
    mm = _mmb if bf else _mm
    interp = mm(s, featsT)
    x = jnp.maximum(mm(interp, w0a[...]) + mm(skipT, w0b[...]) + b0[...], 0.0)
    for i in range(nlayers):
        w, b = refs[7 + 2 * i], refs[8 + 2 * i]
        x = jnp.maximum(mm(x, w[...]) + b[...], 0.0)
    out_x = refs[7 + 2 * nlayers]
    out_x[0] = x
    if wpad:
        n, co = x.shape
        ones = jnp.ones((n, 1), _F32)
        zeros = jnp.zeros((n, wpad - co - 1), _F32)
        refs[8 + 2 * nlayers][0] = jnp.concatenate([x, ones, zeros], axis=1)


def _run_head(denseT, coarse3, featsT, skipT, w0, b0, layers, nblk, wpad, bf=False, nn=None):
    b, nd, _ = denseT.shape
    nc = coarse3.shape[2]
    cfeat = featsT.shape[2]
    blk = nd // nblk
    g = b * nblk
    denseT = denseT.reshape(g, blk, 3)
    skipT = skipT.reshape(g, blk, skipT.shape[2])
    w0a = jnp.transpose(w0[:, :cfeat])
    w0b = jnp.transpose(w0[:, cfeat:])
    co_last = (layers[-1][0] if layers else w0).shape[0]

    def bmap(_g):
        return (_g // nblk, 0, 0)

    if nn is None:
        a0, a1 = denseT, coarse3
        s0 = pl.BlockSpec((1, blk, 3), lambda i: (i, 0, 0))
        s1 = pl.BlockSpec((1, 3, nc), bmap)
    else:
        a0, a1 = nn
        s0 = pl.BlockSpec((1, blk, 3), lambda i: (i, 0, 0))
        s1 = pl.BlockSpec((1, blk, 3), lambda i: (i, 0, 0))
    in_specs = [
        s0,
        s1,
        pl.BlockSpec((1, nc, cfeat), bmap),
        pl.BlockSpec((1, blk, skipT.shape[2]), lambda i: (i, 0, 0)),
        pl.BlockSpec(w0a.shape, lambda i: (0, 0)),
        pl.BlockSpec(w0b.shape, lambda i: (0, 0)),
        pl.BlockSpec(b0.shape, lambda i: (0,)),
    ]
    args = [a0, a1, featsT, skipT, w0a, w0b, b0]
    for (w, bb) in layers:
        args += [jnp.transpose(w), bb]
        in_specs += [pl.BlockSpec((w.shape[1], w.shape[0]), lambda i: (0, 0)),
                     pl.BlockSpec(bb.shape, lambda i: (0,))]
    out_shape = [jax.ShapeDtypeStruct((g, blk, co_last), _F32)]
    out_specs = [pl.BlockSpec((1, blk, co_last), lambda i: (i, 0, 0))]
    if wpad:
        out_shape.append(jax.ShapeDtypeStruct((g, blk, wpad), _F32))
        out_specs.append(pl.BlockSpec((1, blk, wpad), lambda i: (i, 0, 0)))
    outs = pl.pallas_call(
        functools.partial(_head_body, nc, nblk, len(layers), wpad, bf, nn is not None),
        grid=(g,),
        in_specs=in_specs,
        out_specs=out_specs,
        out_shape=out_shape,
    )(*args)
    x = outs[0].reshape(b, nd, co_last)
    if wpad:
        return x, outs[1].reshape(b * nd, wpad)
    return x


def _vid_body(r, p, v, nparts, dense3_ref, out_ref, lout_ref, loc_ref):
    d3 = dense3_ref[0]
    cmin = jnp.min(d3, axis=1, keepdims=True)
    cmax = jnp.max(d3, axis=1, keepdims=True)
    norm = (d3 - cmin) / (cmax - cmin + 1e-8) * (r - 1.0)
    vid = jnp.clip(jnp.round(norm), 0.0, r - 1.0).astype(jnp.int32)
    flat = (vid[0:1, :] * p + (vid[1:2, :] + 1) * (r + 2)
            + vid[2:3, :] + 1)
    bi = pl.program_id(0)
    out_ref[0] = flat + bi * v
    lout_ref[0] = flat
    vp = v // nparts
    for q in range(nparts):
        lid = flat - q * vp
        ok = (lid >= 0) & (lid < vp)
        loc_ref[q, 0] = jnp.where(ok, lid, vp)


def _run_vid(dense3, r, p, nparts):
    b, _, nd = dense3.shape
    v = r * p
    out, lout, loc = pl.pallas_call(
        functools.partial(_vid_body, r, p, v, nparts),
        grid=(b,),
        in_specs=[pl.BlockSpec((1, 3, nd), lambda i: (i, 0, 0))],
        out_specs=[pl.BlockSpec((1, 1, nd), lambda i: (i, 0, 0)),
                   pl.BlockSpec((1, 1, nd), lambda i: (i, 0, 0)),
                   pl.BlockSpec((nparts, 1, 1, nd), lambda i: (0, i, 0, 0))],
        out_shape=[jax.ShapeDtypeStruct((b, 1, nd), jnp.int32),
                   jax.ShapeDtypeStruct((b, 1, nd), jnp.int32),
                   jax.ShapeDtypeStruct((nparts, b, 1, nd), jnp.int32)],
    )(dense3)
    return (out.reshape(b * nd // 128, 128),
            lout.reshape(b, nd // 128, 128),
            loc.reshape(nparts, b, nd // 128, 128))


def _sc_voxelize(rows, loc2, nb, nd, v, nparts, with_counts):
    wp = 128
    vp = v // nparts
    hv = vp + 128
    zr = hv // 16
    npts = nd // 16
    nch = npts // 128
    idxrows = nd // 128
    wr = vp // 16
    nq = nparts // 2
    zsizes = []
    zo = 0
    while zo < zr:
        zsizes.append((zo, min(128, zr - zo)))
        zo += min(128, zr - zo)
    mesh = plsc.VectorSubcoreMesh(core_axis_name="c", subcore_axis_name="s")

    out_type = [jax.ShapeDtypeStruct((nb * v, wp), _F32)]
    scratch = [
        pltpu.VMEM((128, wp), _F32),
        pltpu.VMEM((idxrows, 128), jnp.int32),
        pltpu.VMEM((nch, 128, wp), _F32),
        pltpu.VMEM_SHARED((hv, wp), _F32),
        pltpu.SemaphoreType.DMA,
        pltpu.SemaphoreType.DMA,
        pltpu.SemaphoreType.DMA,
    ]
    if with_counts:
        out_type.append(jax.ShapeDtypeStruct((nb * v, wp), _F32))
        scratch += [pltpu.VMEM((128, wp), _F32),
                    pltpu.VMEM_SHARED((hv, wp), _F32)]

    @functools.partial(pl.kernel, out_type=tuple(out_type), mesh=mesh,
                       scratch_types=scratch)
    def k(rows_hbm, loc_hbm, *rest):
        if with_counts:
            (out_hbm, cout_hbm, zbuf, idxv, rowsv, spmem, zsem, isem, rsem,
             cbuf, cspmem) = rest
        else:
            out_hbm, zbuf, idxv, rowsv, spmem, zsem, isem, rsem = rest
        c = lax.axis_index("c")
        s = lax.axis_index("s")
        zvec = jnp.zeros((16,), _F32)

        def zero_body(i, _):
            for j in range(wp // 16):
                zbuf[i, pl.ds(j * 16, 16)] = zvec
            return _

        lax.fori_loop(0, 128, zero_body, 0)
        if with_counts:
            onevec = jnp.where(lax.iota(jnp.int32, 16) == 0, 1.0, 0.0)

            def cfill_body(i, _):
                cbuf[i, pl.ds(0, 16)] = onevec
                for j in range(1, wp // 16):
                    cbuf[i, pl.ds(j * 16, 16)] = zvec
                return _

            lax.fori_loop(0, 128, cfill_body, 0)
        for b in range(nb):
            base = b * nd + s * npts
            ldh = []
            for j in range(nch):
                ldh.append(pltpu.async_copy(
                    rows_hbm.at[pl.ds(base + j * 128, 128)],
                    rowsv.at[j], rsem))
            for q in range(nq):
                part = q * 2 + c
                pend = []
                for (zo, zs) in zsizes:
                    pend.append(pltpu.async_copy(
                        zbuf.at[pl.ds(0, zs)],
                        spmem.at[pl.ds(s * zr + zo, zs)], zsem))
                    if with_counts:
                        pend.append(pltpu.async_copy(
                            zbuf.at[pl.ds(0, zs)],
                            cspmem.at[pl.ds(s * zr + zo, zs)], zsem))
                pend.append(pltpu.async_copy(
                    loc_hbm.at[pl.ds((part * nb + b) * idxrows, idxrows)],
                    idxv, isem))
                for h in ldh:
                    h.wait()
                ldh = []
                for h in pend:
                    h.wait()
                plsc.subcore_barrier()
                scats = []
                for j in range(nch):
                    scats.append(pltpu.async_copy(
                        rowsv.at[j], spmem.at[idxv.at[s * nch + j]], rsem,
                        add=True))
                    if with_counts:
                        scats.append(pltpu.async_copy(
                            cbuf, cspmem.at[idxv.at[s * nch + j]], rsem,
                            add=True))
                for h in scats:
                    h.wait()
                plsc.subcore_barrier()
                pltpu.sync_copy(
                    spmem.at[pl.ds(s * wr, wr)],
                    out_hbm.at[pl.ds(b * v + part * vp + s * wr, wr)])
                if with_counts:
                    pltpu.sync_copy(
                        cspmem.at[pl.ds(s * wr, wr)],
                        cout_hbm.at[pl.ds(b * v + part * vp + s * wr, wr)])
                plsc.subcore_barrier()

    return k(rows, loc2)


def _sc_gather(table, flat2, m, cw):
    idxrows = m // 128
    nwork = min(32, idxrows)
    nch = idxrows // nwork
    mpw = m // nwork
    mesh = plsc.VectorSubcoreMesh(core_axis_name="c", subcore_axis_name="s")

    @functools.partial(
        pl.kernel,
        out_type=jax.ShapeDtypeStruct((m, cw), _F32),
        mesh=mesh,
        scratch_types=[
            pltpu.VMEM((idxrows, 128), jnp.int32),
            pltpu.VMEM((2, 128, cw), _F32),
            pltpu.SemaphoreType.DMA,
            pltpu.SemaphoreType.DMA,
            pltpu.SemaphoreType.DMA,
            pltpu.SemaphoreType.DMA,
        ],
    )
    def k(table_hbm, flat_hbm, out_hbm, idxv, rowsv, g0, g1, w0, w1):
        c = lax.axis_index("c")
        s = lax.axis_index("s")
        w = s * 2 + c

        @pl.when(w < nwork)
        def _():
            gsem = [g0, g1]
            wsem = [w0, w1]
            pltpu.sync_copy(flat_hbm, idxv)
            gh = [None, None]
            wh = [None, None]
            gh[0] = pltpu.async_copy(table_hbm.at[idxv.at[w * nch]],
                                     rowsv.at[0], gsem[0])
            for j in range(nch):
                sl = j % 2
                nsl = (j + 1) % 2
                if j + 1 < nch:
                    if wh[nsl] is not None:
                        wh[nsl].wait()
                    gh[nsl] = pltpu.async_copy(
                        table_hbm.at[idxv.at[w * nch + j + 1]],
                        rowsv.at[nsl], gsem[nsl])
                gh[sl].wait()
                wh[sl] = pltpu.async_copy(
                    rowsv.at[sl], out_hbm.at[pl.ds(w * mpw + j * 128, 128)],
                    wsem[sl])
            for h in wh:
                if h is not None:
                    h.wait()

    return k(table, flat2)


def _roll_rows(a, sft):
    if sft == 0:
        return a
    n = a.shape[0]
    sft = sft % n
    return jnp.concatenate([a[sft:], a[:sft]], axis=0)


def _conv_body(r, p, ci, co, cnt_col, sep, *refs):
    x = pl.program_id(1)
    nslab = 6 if sep else 3
    wt_ref, vb_ref, out_ref = refs[nslab], refs[nslab + 1], refs[nslab + 2]

    def mean_of(i):
        sl = refs[i][0, 0]
        cnt = refs[i + 3][0, 0] if sep else sl
        m = sl[:, :ci] / jnp.maximum(cnt[:, cnt_col:cnt_col + 1], 1.0)
        return m.astype(jnp.bfloat16)

    fprev = mean_of(0) * jnp.where(x > 0, 1.0, 0.0).astype(jnp.bfloat16)
    fcur = mean_of(1)
    fnxt = mean_of(2) * jnp.where(x < r - 1, 1.0, 0.0).astype(jnp.bfloat16)
    cols = []
    for src in (fprev, fcur, fnxt):
        for dy in (-1, 0, 1):
            for dz in (-1, 0, 1):
                cols.append(_roll_rows(src, dy * (r + 2) + dz))
    a = jnp.concatenate(cols, axis=1)
    acc = _mm(a, wt_ref[...])
    out_ref[0, 0] = jnp.maximum(acc + vb_ref[...], 0.0)


def _run_conv(vox, cnt, cnt_col, vw, vb, r, p, nb, co_pad):
    co, ci = vw.shape[0], vw.shape[1]
    wp = vox.shape[1]
    vox4 = vox.reshape(nb, r, p, wp)
    sep = cnt is not None
    wt = jnp.transpose(vw, (2, 3, 4, 1, 0)).reshape(27 * ci, co)
    if co_pad > co:
        wt = jnp.concatenate(
            [wt, jnp.zeros((27 * ci, co_pad - co), _F32)], axis=1)
        vb = jnp.concatenate([vb, jnp.zeros((co_pad - co,), _F32)])
    wt = wt.astype(jnp.bfloat16)

    def clampm(d):
        def im(b, x):
            return (b, jnp.clip(x + d, 0, r - 1), 0, 0)
        return im

    slab = lambda w: [pl.BlockSpec((1, 1, p, w), clampm(-1)),
                      pl.BlockSpec((1, 1, p, w), clampm(0)),
                      pl.BlockSpec((1, 1, p, w), clampm(1))]
    in_specs = slab(wp)
    args = [vox4, vox4, vox4]
    if sep:
        cnt4 = cnt.reshape(nb, r, p, cnt.shape[1])
        in_specs += slab(cnt.shape[1])
        args += [cnt4, cnt4, cnt4]
    in_specs += [pl.BlockSpec((27 * ci, co_pad), lambda b, x: (0, 0)),
                 pl.BlockSpec((co_pad,), lambda b, x: (0,))]
    args += [wt, vb]
    out = pl.pallas_call(
        functools.partial(_conv_body, r, p, ci, co_pad, cnt_col, sep),
        grid=(nb, r),
        in_specs=in_specs,
        out_specs=pl.BlockSpec((1, 1, p, co_pad), lambda b, x: (b, x, 0, 0)),
        out_shape=jax.ShapeDtypeStruct((nb, r, p, co_pad), _F32),
    )(*args)
    return out.reshape(nb * r * p, co_pad)


def _combine_body(wpad, nhead, co, *refs):
    x = refs[0][0]
    g = refs[1][0][:, :co]
    pwt, pb = refs[2], refs[3]
    y = g + jnp.maximum(_mmb(x, pwt[...]) + pb[...], 0.0)
    out_x = refs[4 + 2 * nhead]
    out_x[0] = y
    if wpad:
        n, co = y.shape
        ones = jnp.ones((n, 1), _F32)
        zeros = jnp.zeros((n, wpad - co - 1), _F32)
        refs[5 + 2 * nhead][0] = jnp.concatenate([y, ones, zeros], axis=1)
    if nhead:
        c0w, c0b = refs[4], refs[5]
        c1w, c1b = refs[6], refs[7]
        r0w, r0b = refs[8], refs[9]
        r1w, r1b = refs[10], refs[11]
        r2w, r2b = refs[12], refs[13]
        h = jnp.maximum(_mmb(y, c0w[...]) + c0b[...], 0.0)
        seg = _mmb(h, c1w[...]) + c1b[...]
        gg = jnp.maximum(_mmb(y, r0w[...]) + r0b[...], 0.0)
        gg = jnp.maximum(_mmb(gg, r1w[...]) + r1b[...], 0.0)
        rec = _mmb(gg, r2w[...]) + r2b[...]
        refs[5 + 2 * nhead][0] = seg
        refs[6 + 2 * nhead][0] = rec


def _run_combine(x, g, pw, pb, wpad=0, heads=None):
    b, nd, ci = x.shape
    co = pw.shape[0]
    gw = g.shape[1]
    g3 = g.reshape(b, nd, gw)
    args = [x, g3, jnp.transpose(pw), pb]
    in_specs = [
        pl.BlockSpec((1, nd, ci), lambda i: (i, 0, 0)),
        pl.BlockSpec((1, nd, gw), lambda i: (i, 0, 0)),
        pl.BlockSpec((ci, co), lambda i: (0, 0)),
        pl.BlockSpec((co,), lambda i: (0,)),
    ]
    nhead = 0
    if heads is not None:
        nhead = 5
        for (w, bb) in heads:
            args += [jnp.transpose(w), bb]
            in_specs += [pl.BlockSpec((w.shape[1], w.shape[0]), lambda i: (0, 0)),
                         pl.BlockSpec(bb.shape, lambda i: (0,))]
    out_shape = [jax.ShapeDtypeStruct((b, nd, co), _F32)]
    out_specs = [pl.BlockSpec((1, nd, co), lambda i: (i, 0, 0))]
    if wpad:
        out_shape.append(jax.ShapeDtypeStruct((b, nd, wpad), _F32))
        out_specs.append(pl.BlockSpec((1, nd, wpad), lambda i: (i, 0, 0)))
    if heads is not None:
        nseg = heads[1][0].shape[0]
        nrec = heads[4][0].shape[0]
        out_shape += [jax.ShapeDtypeStruct((b, nd, nseg), _F32),
                      jax.ShapeDtypeStruct((b, nd, nrec), _F32)]
        out_specs += [pl.BlockSpec((1, nd, nseg), lambda i: (i, 0, 0)),
                      pl.BlockSpec((1, nd, nrec), lambda i: (i, 0, 0))]
    outs = pl.pallas_call(
        functools.partial(_combine_body, wpad, nhead, co),
        grid=(b,),
        in_specs=in_specs,
        out_specs=out_specs,
        out_shape=out_shape,
    )(*args)
    outs = list(outs)
    if wpad:
        outs[1] = outs[1].reshape(b * nd, wpad)
    return outs


def kernel(coords, features, coords_list_0, in_features_0, coords_list_1,
           in_features_1, coords_list_2, in_features_2, coords_list_3,
           in_features_3, coords_list_4, in_features_4,
           s0_fp0_w, s0_fp0_b, s0_blk0_pw, s0_blk0_pb,
           s1_fp0_w, s1_fp0_b, s1_blk0_pw, s1_blk0_pb,
           s2_fp0_w, s2_fp0_b, s2_fp1_w, s2_fp1_b, s2_blk0_pw, s2_blk0_pb,
           s3_fp0_w, s3_fp0_b, s3_fp1_w, s3_fp1_b,
           s3_blk0_pw, s3_blk0_pb, s3_blk0_vw, s3_blk0_vb,
           s3_blk1_pw, s3_blk1_pb, s3_blk1_vw, s3_blk1_vb,
           s4_fp0_w, s4_fp0_b, s4_fp1_w, s4_fp1_b, s4_fp2_w, s4_fp2_b,
           s4_blk0_pw, s4_blk0_pb, s4_blk0_vw, s4_blk0_vb,
           cls0_w, cls0_b, cls1_w, cls1_b,
           rec0_w, rec0_b, rec1_w, rec1_b, rec2_w, rec2_b):
    tt = lambda a: jnp.transpose(a, (0, 2, 1))
    x0 = _run_head(tt(coords_list_4), coords, tt(features), tt(in_features_4),
                   s0_fp0_w, s0_fp0_b, [(s0_blk0_pw, s0_blk0_pb)], 1, 0)
    x1 = _run_head(tt(coords_list_3), coords_list_4, x0, tt(in_features_3),
                   s1_fp0_w, s1_fp0_b, [(s1_blk0_pw, s1_blk0_pb)], 1, 0)
    x2 = _run_head(tt(coords_list_2), coords_list_3, x1, tt(in_features_2),
                   s2_fp0_w, s2_fp0_b,
                   [(s2_fp1_w, s2_fp1_b), (s2_blk0_pw, s2_blk0_pb)], 1, 0)
    x3 = _run_head(tt(coords_list_1), coords_list_2, x2,
                   tt(in_features_1), s3_fp0_w, s3_fp0_b,
                   [(s3_fp1_w, s3_fp1_b)], 1, 0, bf=True)
    _, lflat3, loc3 = _run_vid(coords_list_1, 16, 384, 2)
    for (vw3, vb3, pw3, pb3) in ((s3_blk0_vw, s3_blk0_vb, s3_blk0_pw,
                                  s3_blk0_pb),
                                 (s3_blk1_vw, s3_blk1_vb, s3_blk1_pw,
                                  s3_blk1_pb)):
        gs3 = []
        for b in range(_B):
            locb = loc3[:, b].reshape(-1, 128)
            voxb, cntb = _sc_voxelize(
                x3.reshape(_B * 2048, 128)[b * 2048:(b + 1) * 2048], locb,
                1, 2048, 6144, 2, True)
            convb = _run_conv(voxb, cntb, 0, vw3, vb3, 16, 384, 1, 128)
            gs3.append(_sc_gather(convb, lflat3[b], 2048, 128))
        (x3,) = _run_combine(x3, jnp.concatenate(gs3, axis=0), pw3, pb3)
    x4, rows4 = _run_head(tt(coords_list_0), coords_list_1, x3,
                          tt(in_features_0), s4_fp0_w, s4_fp0_b,
                          [(s4_fp1_w, s4_fp1_b), (s4_fp2_w, s4_fp2_b)], 16, 128, bf=True)
    _, lflat4, loc4 = _run_vid(coords_list_0, 32, 1216, 8)
    gs = []
    for b in range(_B):
        locb = loc4[:, b].reshape(-1, 128)
        (voxb,) = _sc_voxelize(rows4[b * 8192:(b + 1) * 8192], locb,
                               1, 8192, 38912, 8, False)
        convb = _run_conv(voxb, None, 64, s4_blk0_vw, s4_blk0_vb,
                          32, 1216, 1, 128)
        gs.append(_sc_gather(convb, lflat4[b], 8192, 128))
    g = jnp.concatenate(gs, axis=0)
    _, seg, rec = _run_combine(
        x4, g, s4_blk0_pw, s4_blk0_pb, wpad=0,
        heads=[(cls0_w, cls0_b), (cls1_w, cls1_b), (rec0_w, rec0_b),
               (rec1_w, rec1_b), (rec2_w, rec2_b)])
    return tt(seg), tt(rec)
